# baseline (speedup 1.0000x reference)
import functools

import jax
import jax.numpy as jnp
from jax import lax
from jax.experimental import pallas as pl
from jax.experimental.pallas import tpu as pltpu
from jax.experimental.pallas import tpu_sc as plsc


def _sc_gather(user, item, eu_gmf, ei_gmf, eu_mlp, ei_mlp):
    B = user.shape[0]
    FG = eu_gmf.shape[1]
    FM = eu_mlp.shape[1]
    nc, ns = 2, 16
    nw = nc * ns
    bpw = B // nw

    mesh = plsc.VectorSubcoreMesh(core_axis_name="c", subcore_axis_name="s",
                                  num_cores=nc, num_subcores=ns)

    @functools.partial(
        pl.kernel,
        out_type=(
            jax.ShapeDtypeStruct((B, FG), jnp.float32),
            jax.ShapeDtypeStruct((B, FG), jnp.float32),
            jax.ShapeDtypeStruct((B, FM), jnp.float32),
            jax.ShapeDtypeStruct((B, FM), jnp.float32),
        ),
        mesh=mesh,
        compiler_params=pltpu.CompilerParams(use_tc_tiling_on_sc=False),
        scratch_types=[
            pltpu.VMEM((bpw,), jnp.int32),
            pltpu.VMEM((bpw,), jnp.int32),
            pltpu.VMEM((bpw, FG), jnp.float32),
            pltpu.VMEM((bpw, FG), jnp.float32),
            pltpu.VMEM((bpw, FM), jnp.float32),
            pltpu.VMEM((bpw, FM), jnp.float32),
            pltpu.SemaphoreType.DMA,
        ],
    )
    def k(user_hbm, item_hbm, eug_hbm, eig_hbm, eum_hbm, eim_hbm,
          ug_out, ig_out, um_out, im_out,
          uidx, iidx, ug, ig, um, im, sem):
        wid = lax.axis_index("s") * nc + lax.axis_index("c")
        base = wid * bpw
        pltpu.sync_copy(user_hbm.at[pl.ds(base, bpw)], uidx)
        pltpu.sync_copy(item_hbm.at[pl.ds(base, bpw)], iidx)
        c1 = pltpu.async_copy(eug_hbm.at[uidx], ug, sem)
        c2 = pltpu.async_copy(eig_hbm.at[iidx], ig, sem)
        c3 = pltpu.async_copy(eum_hbm.at[uidx], um, sem)
        c4 = pltpu.async_copy(eim_hbm.at[iidx], im, sem)
        c1.wait()
        c2.wait()
        c3.wait()
        c4.wait()
        pltpu.sync_copy(ug, ug_out.at[pl.ds(base, bpw)])
        pltpu.sync_copy(ig, ig_out.at[pl.ds(base, bpw)])
        pltpu.sync_copy(um, um_out.at[pl.ds(base, bpw)])
        pltpu.sync_copy(im, im_out.at[pl.ds(base, bpw)])

    return k(user, item, eu_gmf, ei_gmf, eu_mlp, ei_mlp)


def _tc_dense(ug, ig, um, im, w0a, w0b, b0r, w1t, b1r, w2t, b2r,
              wpg, wph, bpr):
    B = um.shape[0]
    bb = 2048
    nb = B // bb
    FG = ug.shape[1]
    FM = um.shape[1]

    def body(ug_ref, ig_ref, um_ref, im_ref, w0a_ref, w0b_ref, b0_ref,
             w1_ref, b1_ref, w2_ref, b2_ref, wpg_ref, wph_ref, bp_ref,
             out_ref):
        h = um_ref[...] @ w0a_ref[...] + im_ref[...] @ w0b_ref[...]
        h = jnp.maximum(h + b0_ref[...], 0.0)
        h = jnp.maximum(h @ w1_ref[...] + b1_ref[...], 0.0)
        h = jnp.maximum(h @ w2_ref[...] + b2_ref[...], 0.0)
        g = ug_ref[...] * ig_ref[...]
        p = (jnp.sum(g * wpg_ref[...], axis=1)
             + jnp.sum(h * wph_ref[...], axis=1))
        p = p + bp_ref[...]
        out_ref[...] = p.reshape(1, 1, bb)

    full = lambda shp: pl.BlockSpec(shp, lambda i: tuple(0 for _ in shp))
    out3 = pl.pallas_call(
        body,
        grid=(nb,),
        in_specs=[
            pl.BlockSpec((bb, FG), lambda i: (i, 0)),
            pl.BlockSpec((bb, FG), lambda i: (i, 0)),
            pl.BlockSpec((bb, FM), lambda i: (i, 0)),
            pl.BlockSpec((bb, FM), lambda i: (i, 0)),
            full(w0a.shape), full(w0b.shape), full(b0r.shape),
            full(w1t.shape), full(b1r.shape), full(w2t.shape),
            full(b2r.shape), full(wpg.shape), full(wph.shape),
            full(bpr.shape),
        ],
        out_specs=pl.BlockSpec((1, 1, bb), lambda i: (i, 0, 0)),
        out_shape=jax.ShapeDtypeStruct((nb, 1, bb), jnp.float32),
    )(ug, ig, um, im, w0a, w0b, b0r, w1t, b1r, w2t, b2r, wpg, wph, bpr)
    return out3.reshape(B)


def kernel(user, item, eu_gmf, ei_gmf, eu_mlp, ei_mlp,
           W0, b0, W1, b1, W2, b2, Wp, bp):
    FG = eu_gmf.shape[1]
    FM = eu_mlp.shape[1]
    ug, ig, um, im = _sc_gather(user, item, eu_gmf, ei_gmf, eu_mlp, ei_mlp)
    w0a = W0[:, :FM].T
    w0b = W0[:, FM:].T
    w1t = W1.T
    w2t = W2.T
    wpg = Wp[:, :FG]
    wph = Wp[:, FG:]
    return _tc_dense(ug, ig, um, im, w0a, w0b, b0.reshape(1, -1),
                     w1t, b1.reshape(1, -1), w2t, b2.reshape(1, -1),
                     wpg, wph, bp.reshape(1, 1))
